# baseline (speedup 1.0000x reference)
import functools

import jax
import jax.numpy as jnp
from jax import lax
from jax.experimental import pallas as pl
from jax.experimental.pallas import tpu as pltpu
from jax.experimental.pallas import tpu_sc as plsc

_NC = 2
_NS = 16
_NW = _NC * _NS
_K = 128
_JJ = 4


def _sc_aggregate(x, src3, dst3, n_pad):
    n, d = x.shape
    nchunks = src3.shape[1]
    nrounds = nchunks // _JJ
    assert nrounds * _JJ == nchunks and nrounds >= 2
    rows_per_sub = n_pad // _NS

    mesh = plsc.VectorSubcoreMesh(core_axis_name="c", subcore_axis_name="s")

    @functools.partial(
        pl.kernel,
        mesh=mesh,
        out_type=(
            jax.ShapeDtypeStruct((n_pad, d), jnp.float32),
            jax.ShapeDtypeStruct((n_pad, d), jnp.float32),
        ),
        scratch_types=[
            pltpu.VMEM((_JJ, _K), jnp.int32),
            pltpu.VMEM((_JJ, _K), jnp.int32),
            pltpu.VMEM((2, _K, d), jnp.float32),
            pltpu.VMEM_SHARED((n_pad, d), jnp.float32),
        ] + [pltpu.SemaphoreType.DMA] * (2 * _JJ + 4),
    )
    def agg_kernel(x_hbm, src_hbm, dst_hbm, agg0_hbm, agg1_hbm,
                   srcs_v, dsts_v, rows_v, agg_s, *sems):
        issem = sems[:_JJ]
        idsem = sems[_JJ:2 * _JJ]
        gsem = sems[2 * _JJ:2 * _JJ + 2]
        ssem = sems[2 * _JJ + 2:]
        cid = lax.axis_index("c")
        sid = lax.axis_index("s")
        wid = sid * _NC + cid

        def isrc_start(v, c):
            pltpu.async_copy(src_hbm.at[wid, c], srcs_v.at[v], issem[v])

        def isrc_wait(v, c):
            pltpu.make_async_copy(
                src_hbm.at[wid, c], srcs_v.at[v], issem[v]).wait()

        def idst_start(v, c):
            pltpu.async_copy(dst_hbm.at[wid, c], dsts_v.at[v], idsem[v])

        def idst_wait(v, c):
            pltpu.make_async_copy(
                dst_hbm.at[wid, c], dsts_v.at[v], idsem[v]).wait()

        def gather_start(u, v):
            pltpu.async_copy(x_hbm.at[srcs_v.at[v]], rows_v.at[u], gsem[u])

        def gather_wait(u, v):
            pltpu.make_async_copy(
                x_hbm.at[srcs_v.at[v]], rows_v.at[u], gsem[u]).wait()

        def scatter_start(u, v):
            pltpu.async_copy(
                rows_v.at[u], agg_s.at[dsts_v.at[v]], ssem[u], add=True)

        def scatter_wait(u, v):
            pltpu.make_async_copy(
                rows_v.at[u], agg_s.at[dsts_v.at[v]], ssem[u]).wait()

        for j in range(_JJ):
            isrc_start(j, j)
            idst_start(j, j)

        def zbody(i, c):
            for j in range(d // 16):
                rows_v[0, i, pl.ds(j * 16, 16)] = jnp.zeros((16,), jnp.float32)
            return c
        lax.fori_loop(0, _K, zbody, 0)
        for j in range(rows_per_sub // _K):
            pltpu.sync_copy(
                rows_v.at[0], agg_s.at[pl.ds(sid * rows_per_sub + j * _K, _K)])
        plsc.subcore_barrier()

        def round_body(r, carry):
            for j in range(_JJ):
                c = r * _JJ + j
                uj = j % 2

                if j < 2:
                    def step1(j=j, c=c, uj=uj):
                        scatter_wait(uj, (j + 2) % _JJ)
                        idst_start((j + 2) % _JJ, c + 2)
                    pl.when(r > 0)(step1)
                else:
                    scatter_wait(uj, (j + 2) % _JJ)

                    def prefetch_dst(j=j, c=c):
                        idst_start((j + 2) % _JJ, c + 2)
                    pl.when(r < nrounds - 1)(prefetch_dst)

                isrc_wait(j, c)
                gather_start(uj, j)

                def step3(j=j, c=c):
                    up = (j - 1) % 2
                    vp = (j - 1) % _JJ
                    idst_wait(vp, c - 1)
                    gather_wait(up, vp)
                    scatter_start(up, vp)

                def prefetch_src(j=j, c=c):
                    isrc_start((j + 3) % _JJ, c + 3)

                if j == 0:
                    def step3_and_prefetch(step3=step3, prefetch_src=prefetch_src):
                        step3()
                        prefetch_src()
                    pl.when(r > 0)(step3_and_prefetch)
                else:
                    step3()
                    pl.when(r < nrounds - 1)(prefetch_src)
            return carry
        lax.fori_loop(0, nrounds, round_body, 0)

        last = nchunks - 1
        idst_wait((_JJ - 1) % _JJ, last)
        gather_wait(1, _JJ - 1)
        scatter_start(1, _JJ - 1)
        scatter_wait(0, (_JJ - 2) % _JJ)
        scatter_wait(1, _JJ - 1)
        plsc.subcore_barrier()

        wbase = pl.multiple_of(sid * rows_per_sub, _K)

        @pl.when(cid == 0)
        def _():
            pltpu.sync_copy(agg_s.at[pl.ds(wbase, rows_per_sub)],
                            agg0_hbm.at[pl.ds(wbase, rows_per_sub)])

        @pl.when(cid == 1)
        def _():
            pltpu.sync_copy(agg_s.at[pl.ds(wbase, rows_per_sub)],
                            agg1_hbm.at[pl.ds(wbase, rows_per_sub)])

    return agg_kernel(x, src3, dst3)


def _tc_update(x, a0, a1, w, b2d):
    n, d = x.shape

    def body(x_ref, a0_ref, a1_ref, w_ref, b_ref, xo_ref, s_ref):
        m = x_ref[...] + a0_ref[:n] + a1_ref[:n]
        y = jnp.dot(m, w_ref[...], preferred_element_type=jnp.float32)
        y = jnp.maximum(y + b_ref[...], 0.0)
        xo_ref[...] = y
        s_ref[...] = jnp.sum(y, axis=0, keepdims=True)

    return pl.pallas_call(
        body,
        out_shape=(
            jax.ShapeDtypeStruct((n, d), jnp.float32),
            jax.ShapeDtypeStruct((1, d), jnp.float32),
        ),
    )(x, a0, a1, w, b2d)


def kernel(h, edge_index, W1, b1, W2, b2, W3, b3):
    n, d = h.shape
    e = edge_index.shape[1]
    quantum = _NW * _K * _JJ
    epad = ((e + quantum - 1) // quantum) * quantum
    nchunks = epad // (_NW * _K)
    n_pad = ((n + 1 + _NS * _K - 1) // (_NS * _K)) * (_NS * _K)

    src = jnp.concatenate(
        [edge_index[0], jnp.zeros((epad - e,), jnp.int32)])
    dst = jnp.concatenate(
        [edge_index[1], jnp.full((epad - e,), n, jnp.int32)])
    src3 = src.reshape(_NW, nchunks, _K)
    dst3 = dst.reshape(_NW, nchunks, _K)

    x = h
    sums = []
    for (w, b) in ((W1, b1), (W2, b2), (W3, b3)):
        a0, a1 = _sc_aggregate(x, src3, dst3, n_pad)
        x, s = _tc_update(x, a0, a1, w, b.reshape(1, d))
        sums.append(s[0])
    return jnp.concatenate(sums)
